# baseline (speedup 1.0000x reference)
import functools

import jax
import jax.numpy as jnp
from jax import lax
from jax.experimental import pallas as pl
from jax.experimental.pallas import tpu as pltpu
from jax.experimental.pallas import tpu_sc as plsc

_NC, _NS, _L = 2, 16, 16


def _sc_add(B, T, D):
    NW = _NC * _NS
    WT = (T // NW) * D
    W = WT // 2
    NB = 5
    DEPTH = 4
    mesh = plsc.VectorSubcoreMesh(core_axis_name="c", subcore_axis_name="s")

    @functools.partial(
        pl.kernel,
        out_type=jax.ShapeDtypeStruct((B * T * D,), jnp.float32),
        mesh=mesh,
        scratch_types=[
            pltpu.VMEM((WT,), jnp.float32),
            [pltpu.VMEM((W,), jnp.float32) for _ in range(NB)],
            pltpu.SemaphoreType.DMA,
            [pltpu.SemaphoreType.DMA for _ in range(NB)],
            [pltpu.SemaphoreType.DMA for _ in range(NB)],
        ],
    )
    def k(tok_hbm, tab_hbm, out_hbm, tab_v, rbs, stab, sins, souts):
        wid = lax.axis_index("s") * _NC + lax.axis_index("c")
        tbase = wid * WT
        K = 2 * B

        def off(kk):
            sl, b = divmod(kk, B)
            return b * (T * D) + tbase + sl * W

        tab_cp = pltpu.async_copy(tab_hbm.at[pl.ds(tbase, WT)], tab_v, stab)
        in_cp = [None] * K
        out_cp = [None] * K
        for kk in range(DEPTH):
            in_cp[kk] = pltpu.async_copy(
                tok_hbm.at[pl.ds(off(kk), W)], rbs[kk % NB], sins[kk % NB])
        tab_cp.wait()
        for kk in range(K):
            r = kk % NB
            in_cp[kk].wait()
            buf = rbs[r]
            soff = (kk // B) * W

            @plsc.parallel_loop(0, W // _L, unroll=16)
            def _(i, buf=buf, soff=soff):
                buf[pl.ds(i * _L, _L)] = (
                    buf[pl.ds(i * _L, _L)]
                    + tab_v[pl.ds(soff + i * _L, _L)])

            out_cp[kk] = pltpu.async_copy(
                buf, out_hbm.at[pl.ds(off(kk), W)], souts[r])
            nk = kk + DEPTH
            if nk < K:
                nr = nk % NB
                if nk - NB >= 0:
                    out_cp[nk - NB].wait()
                in_cp[nk] = pltpu.async_copy(
                    tok_hbm.at[pl.ds(off(nk), W)], rbs[nr], sins[nr])
        for kk in range(K - NB, K):
            if out_cp[kk] is not None:
                out_cp[kk].wait()

    return k


def kernel(encoded_tokens, pos_table):
    B, T, D = encoded_tokens.shape
    out = _sc_add(B, T, D)(
        encoded_tokens.reshape(-1), pos_table.reshape(-1))
    return out.reshape(B, T, D)


# speedup vs baseline: 1.0182x; 1.0182x over previous
import functools

import jax
import jax.numpy as jnp
from jax import lax
from jax.experimental import pallas as pl
from jax.experimental.pallas import tpu as pltpu
from jax.experimental.pallas import tpu_sc as plsc

_NC, _NS, _L = 2, 16, 16


def _sc_add(B, T, D):
    NW = _NC * _NS
    WT = (T // NW) * D
    W = WT // 2
    NB = 5
    DEPTH = 4
    mesh = plsc.VectorSubcoreMesh(core_axis_name="c", subcore_axis_name="s")

    @functools.partial(
        pl.kernel,
        out_type=jax.ShapeDtypeStruct((B * T * D,), jnp.float32),
        mesh=mesh,
        scratch_types=[
            pltpu.VMEM((WT,), jnp.float32),
            [pltpu.VMEM((W,), jnp.float32) for _ in range(NB)],
            pltpu.SemaphoreType.DMA,
            [pltpu.SemaphoreType.DMA for _ in range(NB)],
            [pltpu.SemaphoreType.DMA for _ in range(NB)],
        ],
    )
    def k(tok_hbm, tab_hbm, out_hbm, tab_v, rbs, stab, sins, souts):
        wid = lax.axis_index("s") * _NC + lax.axis_index("c")
        tbase = wid * WT
        K = 2 * B

        def off(kk):
            sl, b = divmod(kk, B)
            return b * (T * D) + tbase + sl * W

        tab_cp = pltpu.async_copy(tab_hbm.at[pl.ds(tbase, WT)], tab_v, stab)
        in_cp = [None] * K
        out_cp = [None] * K
        for kk in range(DEPTH):
            in_cp[kk] = pltpu.async_copy(
                tok_hbm.at[pl.ds(off(kk), W)], rbs[kk % NB], sins[kk % NB])
        tab_cp.wait()
        for kk in range(K):
            r = kk % NB
            in_cp[kk].wait()
            buf = rbs[r]
            soff = (kk // B) * W

            @plsc.parallel_loop(0, W // _L, unroll=8)
            def _(i, buf=buf, soff=soff):
                buf[pl.ds(i * _L, _L)] = (
                    buf[pl.ds(i * _L, _L)]
                    + tab_v[pl.ds(soff + i * _L, _L)])

            out_cp[kk] = pltpu.async_copy(
                buf, out_hbm.at[pl.ds(off(kk), W)], souts[r])
            nk = kk + DEPTH
            if nk < K:
                nr = nk % NB
                if nk - NB >= 0:
                    out_cp[nk - NB].wait()
                in_cp[nk] = pltpu.async_copy(
                    tok_hbm.at[pl.ds(off(nk), W)], rbs[nr], sins[nr])
        for kk in range(K - NB, K):
            if out_cp[kk] is not None:
                out_cp[kk].wait()

    return k


def kernel(encoded_tokens, pos_table):
    B, T, D = encoded_tokens.shape
    out = _sc_add(B, T, D)(
        encoded_tokens.reshape(-1), pos_table.reshape(-1))
    return out.reshape(B, T, D)
